# baseline (speedup 1.0000x reference)
import jax
import jax.numpy as jnp
from jax.experimental import pallas as pl


def _copy(x_ref, o_ref):
    o_ref[...] = x_ref[...]


@jax.jit
def kernel(x):
    b, c, h, w = x.shape
    n = c * h * w
    x2 = x.reshape(b, n)
    b_blk = 8
    out = pl.pallas_call(
        _copy,
        grid=(b // b_blk,),
        in_specs=[pl.BlockSpec((b_blk, n), lambda i: (i, 0))],
        out_specs=pl.BlockSpec((b_blk, n), lambda i: (i, 0)),
        out_shape=jax.ShapeDtypeStruct((b, n), x.dtype),
    )(x2)
    return out.reshape(b, c, h, w)


# speedup vs baseline: 2.1451x; 2.1451x over previous
import functools

import jax
import jax.numpy as jnp
from jax import lax
from jax.experimental import pallas as pl
from jax.experimental.pallas import tpu as pltpu

_K = 4
_B_BLK = 2


def _body(x_hbm, o_hbm, in_buf, out_buf, in_sem, out_sem,
          *, h: int, w: int, rh: int, n_steps: int):
    def in_copy(i, k):
        return pltpu.make_async_copy(
            x_hbm.at[pl.ds(i * _B_BLK, _B_BLK)], in_buf.at[k], in_sem.at[k])

    def out_copy(i, k):
        return pltpu.make_async_copy(
            out_buf.at[k], o_hbm.at[pl.ds(i * _B_BLK, _B_BLK)], out_sem.at[k])

    for k in range(_K):
        in_copy(k, k).start()

    for i in range(n_steps):
        k = i % _K
        in_copy(i, k).wait()
        if i >= _K:
            out_copy(i - _K, k).wait()

        xb = in_buf[k]
        act = jnp.sum(xb * xb, axis=1)
        lane = lax.broadcasted_iota(jnp.int32, (h, h * w), 1)
        row = lax.broadcasted_iota(jnp.int32, (h, h * w), 0)
        seg = (lane // w) == row
        neg = jnp.float32(-jnp.inf)
        scores = jnp.max(jnp.where(seg[None], act[:, None, :], neg), axis=2)
        gt = (scores[:, None, :] > scores[:, :, None]).astype(jnp.int32)
        rank = jnp.sum(gt, axis=2)
        keep = (rank >= rh).astype(xb.dtype)
        wide = jnp.sum(jnp.where(seg[None], keep[:, :, None],
                                 jnp.float32(0.0)), axis=1)
        out_buf[k] = xb * wide[:, None, :]

        out_copy(i, k).start()
        if i + _K < n_steps:
            in_copy(i + _K, k).start()

    for k in range(_K):
        out_copy(n_steps - _K + k, k).wait()


@jax.jit
def kernel(x):
    b, c, h, w = x.shape
    rh = int(round(0.33 * h))
    n_steps = b // _B_BLK
    x3 = x.reshape(b, c, h * w)
    out = pl.pallas_call(
        functools.partial(_body, h=h, w=w, rh=rh, n_steps=n_steps),
        in_specs=[pl.BlockSpec(memory_space=pltpu.HBM)],
        out_specs=pl.BlockSpec(memory_space=pltpu.HBM),
        out_shape=jax.ShapeDtypeStruct((b, c, h * w), x.dtype),
        scratch_shapes=[
            pltpu.VMEM((_K, _B_BLK, c, h * w), jnp.float32),
            pltpu.VMEM((_K, _B_BLK, c, h * w), jnp.float32),
            pltpu.SemaphoreType.DMA((_K,)),
            pltpu.SemaphoreType.DMA((_K,)),
        ],
    )(x3)
    return out.reshape(b, c, h, w)
